# baseline (device time: 29198 ns/iter reference)
import jax
import jax.numpy as jnp
from jax import lax
from jax.experimental import pallas as pl
from jax.experimental.pallas import tpu as pltpu

N_DEV = 32
N_PLANE = 16
G_PIECE = 4


def _logical(x, y, z):
    return z * 8 + y * 2 + jnp.where(y % 2 == 0, x, 1 - x)


def kernel(x, dy):
    k, d = x.shape
    _, f = dy.shape
    m = d // N_DEV

    def body(
        x_ref,
        dy_ref,
        out_ref,
        acc_ref,
        stage_ref,
        xpack_ref,
        piece_ref,
        fstage_ref,
        comm_ref,
        piece_send_sems,
        piece_recv_sems,
        fwd_send_sems,
        fwd_recv_sems,
    ):
        my = lax.axis_index("i")
        mz = my // 8
        rem = my % 8
        myy = rem // 2
        xp = rem % 2
        mx = jnp.where(myy % 2 == 0, xp, 1 - xp)
        p = mz * 4 + myy
        partner = _logical(1 - mx, myy, mz)

        barrier = pltpu.get_barrier_semaphore()
        for o in range(1, N_DEV):
            pl.semaphore_signal(
                barrier,
                inc=1,
                device_id=((my + o) % N_DEV,),
                device_id_type=pl.DeviceIdType.MESH,
            )

        with jax.named_scope("dot"):
            acc_ref[...] = lax.dot_general(
                x_ref[...].astype(jnp.bfloat16),
                dy_ref[...].astype(jnp.bfloat16),
                dimension_numbers=(((0,), (0,)), ((), ())),
                preferred_element_type=jnp.float32,
            )
            stage_ref[...] = acc_ref[...].astype(jnp.bfloat16)

        with jax.named_scope("bwait"):
            pl.semaphore_wait(barrier, N_DEV - 1)

        def plane_member(q, layer_x):
            return _logical(layer_x, q % 4, q // 4)

        with jax.named_scope("pack"):
            for o in range(1, N_PLANE + 1):
                q = (p + o) % N_PLANE
                owner = plane_member(q, 1 - mx)
                xpack_ref[o - 1] = stage_ref[pl.ds(owner * m, m), :]

        group = N_PLANE // G_PIECE
        with jax.named_scope("piece_issue"):
            piece_rdmas = []
            for g in range(G_PIECE):
                rdma = pltpu.make_async_remote_copy(
                    src_ref=xpack_ref.at[pl.ds(g * group, group)],
                    dst_ref=piece_ref.at[pl.ds(g * group, group)],
                    send_sem=piece_send_sems.at[g],
                    recv_sem=piece_recv_sems.at[g],
                    device_id=(partner,),
                    device_id_type=pl.DeviceIdType.MESH,
                )
                rdma.start()
                piece_rdmas.append(rdma)

        with jax.named_scope("fwd_stream"):
            fwd_rdmas = []
            for o in range(1, N_PLANE):
                q = (p + o) % N_PLANE
                owner = plane_member(q, mx)
                if (o - 1) % group == 0:
                    piece_rdmas[(o - 1) // group].wait_recv()
                fstage_ref[o - 1] = (
                    piece_ref[o - 1] + stage_ref[pl.ds(owner * m, m), :]
                )
                rdma = pltpu.make_async_remote_copy(
                    src_ref=fstage_ref.at[o - 1],
                    dst_ref=comm_ref.at[o - 1],
                    send_sem=fwd_send_sems.at[o - 1],
                    recv_sem=fwd_recv_sems.at[o - 1],
                    device_id=(owner,),
                    device_id_type=pl.DeviceIdType.MESH,
                )
                rdma.start()
                fwd_rdmas.append(rdma)

        with jax.named_scope("own_chunk"):
            out_ref[...] = (
                acc_ref[pl.ds(my * m, m), :]
                + piece_ref[N_PLANE - 1].astype(jnp.float32)
            )
        with jax.named_scope("gather"):
            for o in range(1, N_PLANE):
                fwd_rdmas[o - 1].wait_recv()
                out_ref[...] += comm_ref[o - 1].astype(jnp.float32)

        with jax.named_scope("drain"):
            for r in piece_rdmas:
                r.wait_send()
            for r in fwd_rdmas:
                r.wait_send()

    return pl.pallas_call(
        body,
        out_shape=jax.ShapeDtypeStruct((m, f), jnp.float32),
        in_specs=[
            pl.BlockSpec(memory_space=pltpu.VMEM),
            pl.BlockSpec(memory_space=pltpu.VMEM),
        ],
        out_specs=pl.BlockSpec(memory_space=pltpu.VMEM),
        scratch_shapes=[
            pltpu.VMEM((d, f), jnp.float32),
            pltpu.VMEM((d, f), jnp.bfloat16),
            pltpu.VMEM((N_PLANE, m, f), jnp.bfloat16),
            pltpu.VMEM((N_PLANE, m, f), jnp.bfloat16),
            pltpu.VMEM((N_PLANE - 1, m, f), jnp.bfloat16),
            pltpu.VMEM((N_PLANE - 1, m, f), jnp.bfloat16),
            pltpu.SemaphoreType.DMA((G_PIECE,)),
            pltpu.SemaphoreType.DMA((G_PIECE,)),
            pltpu.SemaphoreType.DMA((N_PLANE - 1,)),
            pltpu.SemaphoreType.DMA((N_PLANE - 1,)),
        ],
        compiler_params=pltpu.CompilerParams(collective_id=0),
    )(x, dy)


# device time: 28162 ns/iter; 1.0368x vs baseline; 1.0368x over previous
import jax
import jax.numpy as jnp
from jax import lax
from jax.experimental import pallas as pl
from jax.experimental.pallas import tpu as pltpu

N_DEV = 32
N_PLANE = 16


def _logical(x, y, z):
    return z * 8 + y * 2 + jnp.where(y % 2 == 0, x, 1 - x)


def kernel(x, dy):
    k, d = x.shape
    _, f = dy.shape
    m = d // N_DEV

    def body(
        x_ref,
        dy_ref,
        out_ref,
        acc_ref,
        stage_ref,
        piece_ref,
        fstage_ref,
        comm_ref,
        piece_send_sems,
        piece_recv_sems,
        fwd_send_sems,
        fwd_recv_sems,
    ):
        my = lax.axis_index("i")
        mz = my // 8
        rem = my % 8
        myy = rem // 2
        xp = rem % 2
        mx = jnp.where(myy % 2 == 0, xp, 1 - xp)
        p = mz * 4 + myy
        partner = _logical(1 - mx, myy, mz)

        barrier = pltpu.get_barrier_semaphore()
        for o in range(1, N_DEV):
            pl.semaphore_signal(
                barrier,
                inc=1,
                device_id=((my + o) % N_DEV,),
                device_id_type=pl.DeviceIdType.MESH,
            )

        with jax.named_scope("dot"):
            acc_ref[...] = lax.dot_general(
                x_ref[...].astype(jnp.bfloat16),
                dy_ref[...].astype(jnp.bfloat16),
                dimension_numbers=(((0,), (0,)), ((), ())),
                preferred_element_type=jnp.float32,
            )
            stage_ref[...] = acc_ref[...].astype(jnp.bfloat16)

        with jax.named_scope("bwait"):
            pl.semaphore_wait(barrier, N_DEV - 1)

        def plane_member(q, layer_x):
            return _logical(layer_x, q % 4, q // 4)

        with jax.named_scope("piece_issue"):
            piece_rdmas = []
            for o in range(1, N_PLANE + 1):
                q = (p + o) % N_PLANE
                owner = plane_member(q, 1 - mx)
                rdma = pltpu.make_async_remote_copy(
                    src_ref=stage_ref.at[pl.ds(owner * m, m), :],
                    dst_ref=piece_ref.at[o - 1],
                    send_sem=piece_send_sems.at[o - 1],
                    recv_sem=piece_recv_sems.at[o - 1],
                    device_id=(partner,),
                    device_id_type=pl.DeviceIdType.MESH,
                )
                rdma.start()
                piece_rdmas.append(rdma)

        with jax.named_scope("fwd_stream"):
            fwd_rdmas = []
            for o in range(1, N_PLANE):
                q = (p + o) % N_PLANE
                owner = plane_member(q, mx)
                piece_rdmas[o - 1].wait_recv()
                fstage_ref[o - 1] = (
                    piece_ref[o - 1] + stage_ref[pl.ds(owner * m, m), :]
                )
                rdma = pltpu.make_async_remote_copy(
                    src_ref=fstage_ref.at[o - 1],
                    dst_ref=comm_ref.at[o - 1],
                    send_sem=fwd_send_sems.at[o - 1],
                    recv_sem=fwd_recv_sems.at[o - 1],
                    device_id=(owner,),
                    device_id_type=pl.DeviceIdType.MESH,
                )
                rdma.start()
                fwd_rdmas.append(rdma)

        with jax.named_scope("own_chunk"):
            piece_rdmas[N_PLANE - 1].wait_recv()
            out_ref[...] = (
                acc_ref[pl.ds(my * m, m), :]
                + piece_ref[N_PLANE - 1].astype(jnp.float32)
            )
        with jax.named_scope("gather"):
            for o in range(1, N_PLANE):
                fwd_rdmas[o - 1].wait_recv()
                out_ref[...] += comm_ref[o - 1].astype(jnp.float32)

        with jax.named_scope("drain"):
            for r in piece_rdmas:
                r.wait_send()
            for r in fwd_rdmas:
                r.wait_send()

    return pl.pallas_call(
        body,
        out_shape=jax.ShapeDtypeStruct((m, f), jnp.float32),
        in_specs=[
            pl.BlockSpec(memory_space=pltpu.VMEM),
            pl.BlockSpec(memory_space=pltpu.VMEM),
        ],
        out_specs=pl.BlockSpec(memory_space=pltpu.VMEM),
        scratch_shapes=[
            pltpu.VMEM((d, f), jnp.float32),
            pltpu.VMEM((d, f), jnp.bfloat16),
            pltpu.VMEM((N_PLANE, m, f), jnp.bfloat16),
            pltpu.VMEM((N_PLANE - 1, m, f), jnp.bfloat16),
            pltpu.VMEM((N_PLANE - 1, m, f), jnp.bfloat16),
            pltpu.SemaphoreType.DMA((N_PLANE,)),
            pltpu.SemaphoreType.DMA((N_PLANE,)),
            pltpu.SemaphoreType.DMA((N_PLANE - 1,)),
            pltpu.SemaphoreType.DMA((N_PLANE - 1,)),
        ],
        compiler_params=pltpu.CompilerParams(collective_id=0),
    )(x, dy)
